# baseline (speedup 1.0000x reference)
import functools

import jax
import jax.numpy as jnp
from jax import lax
from jax.experimental import pallas as pl
from jax.experimental.pallas import tpu as pltpu
from jax.experimental.pallas import tpu_sc as plsc

EMBED = 256
HEADS = 8
LEVELS = 4
POINTS = 8
CAMS = 6
NQ = 2500
D = 4
SHAPES = ((50, 80), (25, 40), (13, 20), (7, 10))
LSTART = (0, 4000, 5000, 5260)
L = 5330
HD = EMBED // HEADS

QP = 2560
QB = 128
NW = 32
QPW = QP // NW
IDX_PER_Q = CAMS * HEADS * LEVELS * POINTS
WGT_PER_Q = IDX_PER_Q * 4
CHUNK = LEVELS * POINTS
NCHUNK = CAMS * HEADS
SEG = tuple(h * w + w + 1 for h, w in SHAPES)
LOFF = (0, SEG[0], SEG[0] + SEG[1], SEG[0] + SEG[1] + SEG[2])
PLANE = sum(SEG)


def _vproj_body(x_ref, w_ref, b_ref, o_ref):
    o_ref[...] = jnp.dot(x_ref[...], w_ref[...],
                         preferred_element_type=jnp.float32) + b_ref[...]


def _vproj(x, w, b):
    n = x.shape[0]
    blk = 512
    grid = (n + blk - 1) // blk
    return pl.pallas_call(
        _vproj_body,
        grid=(grid,),
        in_specs=[
            pl.BlockSpec((blk, EMBED), lambda i: (i, 0)),
            pl.BlockSpec((EMBED, EMBED), lambda i: (0, 0)),
            pl.BlockSpec((1, EMBED), lambda i: (0, 0)),
        ],
        out_specs=pl.BlockSpec((blk, EMBED), lambda i: (i, 0)),
        out_shape=jax.ShapeDtypeStruct((n, EMBED), jnp.float32),
    )(x, w, b)


def _expand_body(q_ref, reft_ref, mk_ref, woff_ref, boff_ref, wattn_ref,
                 battn_ref, selh_ref, base_ref, idx_ref, w0_ref, w1_ref,
                 w2_ref, w3_ref, m_ref):
    q = q_ref[...]
    off = jnp.dot(q, woff_ref[...],
                  preferred_element_type=jnp.float32) + boff_ref[...]
    logits = jnp.dot(q, wattn_ref[...],
                     preferred_element_type=jnp.float32) + battn_ref[...]
    selh = selh_ref[...]
    mxh = []
    for h in range(HEADS):
        m01 = jnp.maximum(logits[:, h * 8:h * 8 + 8],
                          logits[:, 64 + h * 8:64 + h * 8 + 8])
        m23 = jnp.maximum(logits[:, 128 + h * 8:128 + h * 8 + 8],
                          logits[:, 192 + h * 8:192 + h * 8 + 8])
        mxh.append(jnp.max(jnp.maximum(m01, m23), axis=1, keepdims=True))
    mx8 = jnp.concatenate(mxh, axis=1)
    e = jnp.exp(logits - jnp.dot(mx8, selh,
                                 preferred_element_type=jnp.float32))
    s8 = jnp.dot(e, selh_ref[...].T,
                 preferred_element_type=jnp.float32)
    aw = e * jnp.dot(1.0 / s8, selh,
                     preferred_element_type=jnp.float32)
    mk = mk_ref[...]
    reft = reft_ref[...]
    base0 = base_ref[...]
    mcols = []
    wxy = (None, None, None, None)
    for cam in range(CAMS):
        msum = jnp.sum(mk[:, cam * D:(cam + 1) * D], axis=1, keepdims=True)
        mcam = (msum > 0.0).astype(jnp.float32)
        mcols.append(mcam)
        refx = reft[:, cam * 64:(cam + 1) * 64]
        refy = reft[:, 384 + cam * 64:384 + (cam + 1) * 64]
        for lvl in range(LEVELS):
            Hl, Wl = SHAPES[lvl]
            x = refx * Wl + off[:, lvl * 64:(lvl + 1) * 64] - 0.5
            y = refy * Hl + off[:, 256 + lvl * 64:256 + (lvl + 1) * 64] - 0.5
            x0 = jnp.floor(x)
            y0 = jnp.floor(y)
            wx1 = x - x0
            wy1 = y - y0
            wbase = aw[:, lvl * 64:(lvl + 1) * 64] * mcam
            basei = base0 + (cam * HEADS * PLANE + LOFF[lvl] + Wl + 1)
            x0c = jnp.clip(x0, -1.0, Wl - 1).astype(jnp.int32)
            y0c = jnp.clip(y0, -1.0, Hl - 1).astype(jnp.int32)
            idx = basei + y0c * Wl + x0c
            ri = (cam * LEVELS + lvl) * 64
            idx_ref[:, ri:ri + 64] = idx
            wrefs = (w0_ref, w1_ref, w2_ref, w3_ref)
            for tap, (dy, dx) in enumerate(((0, 0), (0, 1), (1, 0), (1, 1))):
                ixf = x0 + dx
                iyf = y0 + dy
                valid = ((ixf >= 0.0) & (ixf <= Wl - 1)
                         & (iyf >= 0.0) & (iyf <= Hl - 1))
                wt = wbase * (wx1 if dx else 1.0 - wx1) \
                    * (wy1 if dy else 1.0 - wy1) * valid.astype(jnp.float32)
                wrefs[tap][:, ri:ri + 64] = wt
    m_ref[...] = jnp.concatenate(mcols, axis=1)


def _expand(qp, reft, maskq, woffp, boffp, wattnp, battnp, selh, base_row):
    grid = QP // QB
    wspec = pl.BlockSpec((QB, IDX_PER_Q), lambda i: (i, 0))
    wshape = jax.ShapeDtypeStruct((QP, IDX_PER_Q), jnp.float32)
    return pl.pallas_call(
        _expand_body,
        grid=(grid,),
        in_specs=[
            pl.BlockSpec((QB, EMBED), lambda i: (i, 0)),
            pl.BlockSpec((QB, 768), lambda i: (i, 0)),
            pl.BlockSpec((QB, 24), lambda i: (i, 0)),
            pl.BlockSpec((EMBED, 512), lambda i: (0, 0)),
            pl.BlockSpec((1, 512), lambda i: (0, 0)),
            pl.BlockSpec((EMBED, EMBED), lambda i: (0, 0)),
            pl.BlockSpec((1, EMBED), lambda i: (0, 0)),
            pl.BlockSpec((HEADS, EMBED), lambda i: (0, 0)),
            pl.BlockSpec((1, 64), lambda i: (0, 0)),
        ],
        out_specs=[
            pl.BlockSpec((QB, IDX_PER_Q), lambda i: (i, 0)),
            wspec, wspec, wspec, wspec,
            pl.BlockSpec((QB, CAMS), lambda i: (i, 0)),
        ],
        out_shape=[
            jax.ShapeDtypeStruct((QP, IDX_PER_Q), jnp.int32),
            wshape, wshape, wshape, wshape,
            jax.ShapeDtypeStruct((QP, CAMS), jnp.float32),
        ],
    )(qp, reft, maskq, woffp, boffp, wattnp, battnp, selh, base_row)


def _sc_gather(v_t, idx_q, wgt_q):
    mesh = plsc.VectorSubcoreMesh(core_axis_name="c", subcore_axis_name="s")
    BIG = 128
    NBIG = IDX_PER_Q // BIG
    DEPTH = 4

    @functools.partial(
        pl.kernel,
        mesh=mesh,
        out_type=jax.ShapeDtypeStruct((QP, EMBED), jnp.float32),
        scratch_types=(
            [pltpu.VMEM((IDX_PER_Q,), jnp.int32)] * 2
            + [pltpu.VMEM((WGT_PER_Q,), jnp.float32)] * 2
            + [pltpu.VMEM((BIG, 128), jnp.float32)] * DEPTH
            + [pltpu.VMEM((EMBED,), jnp.float32)]
            + [pltpu.SemaphoreType.DMA] * (DEPTH + 2)
        ),
    )
    def body(v_hbm, idx_hbm, wgt_hbm, out_hbm, idx_v0, idx_v1, wgt_v0,
             wgt_v1, b0, b1, b2, b3, out_v, g0, g1, g2, g3, si0, si1):
        wid = lax.axis_index("s") * 2 + lax.axis_index("c")
        q0 = wid * QPW
        bufs = (b0, b1, b2, b3)
        gsems = (g0, g1, g2, g3)
        isets = ((idx_v0, wgt_v0, si0), (idx_v1, wgt_v1, si1))
        zero = jnp.zeros((16,), jnp.float32)

        def process_big(wgt_v, bc, buf):
            for g in range(BIG // 64):
                goff = (bc * 2 + g) * 64
                roff = g * 64

                def head_body(hh, carry):
                    woff = (goff + hh * 8) * 4

                    def mac(i, accs):
                        a = list(accs)
                        r0 = roff + hh * 8 + i * 4
                        wv = wgt_v[pl.ds(woff + i * 16, 16)]
                        for k in range(4):
                            for t in range(4):
                                w = wv[k * 4 + t]
                                a[2 * t] = a[2 * t] \
                                    + w * buf[r0 + k, pl.ds(32 * t, 16)]
                                a[2 * t + 1] = a[2 * t + 1] \
                                    + w * buf[r0 + k, pl.ds(32 * t + 16, 16)]
                        return tuple(a)

                    accs = lax.fori_loop(0, 2, mac, (zero,) * 8)
                    acc0 = accs[0] + accs[2] + accs[4] + accs[6]
                    acc1 = accs[1] + accs[3] + accs[5] + accs[7]
                    s0 = hh * HD
                    out_v[pl.ds(s0, 16)] = out_v[pl.ds(s0, 16)] + acc0
                    out_v[pl.ds(s0 + 16, 16)] = \
                        out_v[pl.ds(s0 + 16, 16)] + acc1
                    return carry

                lax.fori_loop(0, HEADS, head_body, 0)

        def fetch(gq, iset):
            idx_v, wgt_v, sem = iset
            pltpu.async_copy(idx_hbm.at[gq], idx_v, sem)
            pltpu.async_copy(wgt_hbm.at[gq], wgt_v, sem)

        def drain(iset):
            idx_v, wgt_v, sem = iset
            pltpu.make_async_copy(idx_hbm.at[0], idx_v, sem).wait()
            pltpu.make_async_copy(wgt_hbm.at[0], wgt_v, sem).wait()

        def do_q(gq, iset):
            idx_v, wgt_v, _ = iset
            for j in range(EMBED // 16):
                out_v[pl.ds(j * 16, 16)] = zero
            handles = []
            for bc in range(NBIG):
                handles.append(pltpu.async_copy(
                    v_hbm.at[idx_v.at[pl.ds(bc * BIG, BIG)]],
                    bufs[bc % DEPTH], gsems[bc % DEPTH]))
                if bc >= DEPTH - 1:
                    handles[bc - DEPTH + 1].wait()
                    process_big(wgt_v, bc - DEPTH + 1,
                                bufs[(bc - DEPTH + 1) % DEPTH])
            for bc in range(NBIG - DEPTH + 1, NBIG):
                handles[bc].wait()
                process_big(wgt_v, bc, bufs[bc % DEPTH])
            pltpu.sync_copy(out_v, out_hbm.at[gq])

        fetch(q0, isets[0])

        def per_pair(i, carry):
            qa = q0 + 2 * i
            drain(isets[0])
            fetch(qa + 1, isets[1])
            do_q(qa, isets[0])
            drain(isets[1])
            fetch(jnp.minimum(qa + 2, QP - 1), isets[0])
            do_q(qa + 1, isets[1])
            return carry

        lax.fori_loop(0, QPW // 2, per_pair, 0)
        drain(isets[0])

    return body(v_t, idx_q, wgt_q)


def _outproj_body(s_ref, m_ref, q_ref, w_ref, b_ref, o_ref):
    cnt = jnp.maximum(jnp.sum(m_ref[...], axis=1, keepdims=True), 1.0)
    s = s_ref[...] / cnt
    o_ref[...] = jnp.dot(s, w_ref[...],
                         preferred_element_type=jnp.float32) \
        + b_ref[...] + q_ref[...]


def _outproj(slots, mq, qpad, w, b):
    blk = 512
    grid = QP // blk
    return pl.pallas_call(
        _outproj_body,
        grid=(grid,),
        in_specs=[
            pl.BlockSpec((blk, EMBED), lambda i: (i, 0)),
            pl.BlockSpec((blk, CAMS), lambda i: (i, 0)),
            pl.BlockSpec((blk, EMBED), lambda i: (i, 0)),
            pl.BlockSpec((EMBED, EMBED), lambda i: (0, 0)),
            pl.BlockSpec((1, EMBED), lambda i: (0, 0)),
        ],
        out_specs=pl.BlockSpec((blk, EMBED), lambda i: (i, 0)),
        out_shape=jax.ShapeDtypeStruct((QP, EMBED), jnp.float32),
    )(slots, mq, qpad, w, b)


def kernel(query, key, value, reference_points_cam, bev_mask, spatial_shapes,
           level_start_index, W_off, b_off, W_attn, b_attn, W_val, b_val,
           W_out, b_out):
    q2 = query[0]
    v3 = value.reshape(CAMS * L, EMBED)

    vproj = _vproj(v3, W_val, b_val.reshape(1, EMBED))
    v5 = vproj.reshape(CAMS, L, HEADS, HD).transpose(0, 2, 1, 3)
    vp = jnp.pad(v5, ((0, 0), (0, 0), (96, 96), (0, 0)))
    segs = []
    for lvl, (Hl, Wl) in enumerate(SHAPES):
        start = 96 + LSTART[lvl] - (Wl + 1)
        segs.append(jnp.concatenate(
            [vp[:, :, start + sh:start + sh + SEG[lvl], :]
             for sh in (0, 1, Wl, Wl + 1)], axis=-1))
    v4 = jnp.concatenate(segs, axis=2) \
        .reshape(CAMS * HEADS * PLANE, 4 * HD)

    pad = QP - NQ
    qp = jnp.pad(q2, ((0, pad), (0, 0)))
    rpc = reference_points_cam[:, 0]
    reft = jnp.pad(
        jnp.tile(rpc.transpose(1, 3, 0, 2), (1, 1, 1, 16)).reshape(NQ, 768),
        ((0, pad), (0, 0)))
    maskq = jnp.pad(
        bev_mask[:, 0].astype(jnp.float32).transpose(1, 0, 2).reshape(NQ, 24),
        ((0, pad), (0, 0)))
    woffp = W_off.reshape(EMBED, HEADS, LEVELS, POINTS, 2) \
        .transpose(0, 4, 2, 1, 3).reshape(EMBED, 512)
    boffp = b_off.reshape(HEADS, LEVELS, POINTS, 2) \
        .transpose(3, 1, 0, 2).reshape(1, 512)
    wattnp = W_attn.reshape(EMBED, HEADS, LEVELS, POINTS) \
        .transpose(0, 2, 1, 3).reshape(EMBED, EMBED)
    battnp = b_attn.reshape(HEADS, LEVELS, POINTS) \
        .transpose(1, 0, 2).reshape(1, EMBED)
    cols = jnp.arange(EMBED, dtype=jnp.int32)
    selh = ((cols[None, :] // POINTS) % HEADS
            == jnp.arange(HEADS, dtype=jnp.int32)[:, None]) \
        .astype(jnp.float32)
    base_row = (jnp.repeat(jnp.arange(HEADS, dtype=jnp.int32), POINTS)
                * PLANE).reshape(1, 64)

    idx_q, w0, w1, w2, w3, m_q = _expand(
        qp, reft, maskq, woffp, boffp, wattnp, battnp, selh, base_row)
    wgt_q = jnp.stack([w0, w1, w2, w3], axis=-1).reshape(QP, WGT_PER_Q)

    slots = _sc_gather(v4, idx_q, wgt_q)

    out = _outproj(slots, m_q, qp, W_out, b_out.reshape(1, EMBED))
    return out[:NQ].reshape(1, NQ, EMBED)


# speedup vs baseline: 1.8487x; 1.8487x over previous
import functools

import jax
import jax.numpy as jnp
from jax import lax
from jax.experimental import pallas as pl
from jax.experimental.pallas import tpu as pltpu
from jax.experimental.pallas import tpu_sc as plsc

EMBED = 256
HEADS = 8
LEVELS = 4
POINTS = 8
CAMS = 6
NQ = 2500
D = 4
SHAPES = ((50, 80), (25, 40), (13, 20), (7, 10))
LSTART = (0, 4000, 5000, 5260)
L = 5330
HD = EMBED // HEADS

QP = 2560
QB = 128
NW = 32
QPW = QP // NW
IDX_PER_Q = CAMS * HEADS * LEVELS * POINTS
WGT_PER_Q = IDX_PER_Q * 4
CHUNK = LEVELS * POINTS
NCHUNK = CAMS * HEADS
SEG = (4081, 1041, 281, 85)
LOFF = (0, SEG[0], SEG[0] + SEG[1], SEG[0] + SEG[1] + SEG[2])
PLANE = sum(SEG)
PPAD = 96


def _vproj_body(x_ref, w_ref, b_ref, o_ref):
    o_ref[...] = jnp.dot(x_ref[0, :, 0, :], w_ref[...],
                         preferred_element_type=jnp.float32)[None] \
        + b_ref[...]


def _vproj(x4, w, b):
    return pl.pallas_call(
        _vproj_body,
        grid=(CAMS,),
        in_specs=[
            pl.BlockSpec((1, L, 1, EMBED), lambda c: (c, 0, 0, 0)),
            pl.BlockSpec((EMBED, EMBED), lambda c: (0, 0)),
            pl.BlockSpec((1, EMBED), lambda c: (0, 0)),
        ],
        out_specs=pl.BlockSpec((1, L, EMBED), lambda c: (c, 0, 0)),
        out_shape=jax.ShapeDtypeStruct((CAMS, L, EMBED), jnp.float32),
    )(x4, w, b)


def _pack_body(vp_ref, o_ref):
    for lvl in range(LEVELS):
        Hl, Wl = SHAPES[lvl]
        start = PPAD + LSTART[lvl] - (Wl + 1)
        n = SEG[lvl]
        for h in range(4):
            pieces = [vp_ref[0, start + sh:start + sh + n,
                             h * HD:(h + 1) * HD]
                      for sh in (0, 1, Wl, Wl + 1)]
            o_ref[0, h, LOFF[lvl]:LOFF[lvl] + n, :] = \
                jnp.concatenate(pieces, axis=1)


def _pack(vp2):
    return pl.pallas_call(
        _pack_body,
        grid=(CAMS, 2),
        in_specs=[pl.BlockSpec((1, L + 2 * PPAD, 128),
                               lambda c, hb: (c, 0, hb))],
        out_specs=pl.BlockSpec((1, 4, PLANE, 128), lambda c, hb: (c, hb, 0, 0)),
        out_shape=jax.ShapeDtypeStruct((CAMS, HEADS, PLANE, 128),
                                       jnp.float32),
    )(vp2)


def _expand_body(q_ref, reft_ref, mk_ref, woff_ref, boff_ref, wattn_ref,
                 battn_ref, selh_ref, base_ref, idx_ref, w0_ref, w1_ref,
                 w2_ref, w3_ref, m_ref):
    q = q_ref[...]
    off = jnp.dot(q, woff_ref[...],
                  preferred_element_type=jnp.float32) + boff_ref[...]
    logits = jnp.dot(q, wattn_ref[...],
                     preferred_element_type=jnp.float32) + battn_ref[...]
    selh = selh_ref[...]
    mxh = []
    for h in range(HEADS):
        m01 = jnp.maximum(logits[:, h * 8:h * 8 + 8],
                          logits[:, 64 + h * 8:64 + h * 8 + 8])
        m23 = jnp.maximum(logits[:, 128 + h * 8:128 + h * 8 + 8],
                          logits[:, 192 + h * 8:192 + h * 8 + 8])
        mxh.append(jnp.max(jnp.maximum(m01, m23), axis=1, keepdims=True))
    mx8 = jnp.concatenate(mxh, axis=1)
    e = jnp.exp(logits - jnp.dot(mx8, selh,
                                 preferred_element_type=jnp.float32))
    s8 = jnp.dot(e, selh_ref[...].T,
                 preferred_element_type=jnp.float32)
    aw = e * jnp.dot(1.0 / s8, selh,
                     preferred_element_type=jnp.float32)
    mk = mk_ref[...]
    reft = reft_ref[...]
    base0 = base_ref[...]
    mcols = []
    wxy = (None, None, None, None)
    for cam in range(CAMS):
        msum = jnp.sum(mk[:, cam * D:(cam + 1) * D], axis=1, keepdims=True)
        mcam = (msum > 0.0).astype(jnp.float32)
        mcols.append(mcam)
        refx = reft[:, cam * 64:(cam + 1) * 64]
        refy = reft[:, 384 + cam * 64:384 + (cam + 1) * 64]
        for lvl in range(LEVELS):
            Hl, Wl = SHAPES[lvl]
            x = refx * Wl + off[:, lvl * 64:(lvl + 1) * 64] - 0.5
            y = refy * Hl + off[:, 256 + lvl * 64:256 + (lvl + 1) * 64] - 0.5
            x0 = jnp.floor(x)
            y0 = jnp.floor(y)
            wx1 = x - x0
            wy1 = y - y0
            wbase = aw[:, lvl * 64:(lvl + 1) * 64] * mcam
            basei = base0 + (cam * HEADS * PLANE + LOFF[lvl] + Wl + 1)
            x0c = jnp.clip(x0, -1.0, Wl - 1).astype(jnp.int32)
            y0c = jnp.clip(y0, -1.0, Hl - 1).astype(jnp.int32)
            idx = basei + y0c * Wl + x0c
            ri = (cam * LEVELS + lvl) * 64
            idx_ref[:, ri:ri + 64] = idx
            wrefs = (w0_ref, w1_ref, w2_ref, w3_ref)
            for tap, (dy, dx) in enumerate(((0, 0), (0, 1), (1, 0), (1, 1))):
                ixf = x0 + dx
                iyf = y0 + dy
                valid = ((ixf >= 0.0) & (ixf <= Wl - 1)
                         & (iyf >= 0.0) & (iyf <= Hl - 1))
                wt = wbase * (wx1 if dx else 1.0 - wx1) \
                    * (wy1 if dy else 1.0 - wy1) * valid.astype(jnp.float32)
                wrefs[tap][:, ri:ri + 64] = wt
    m_ref[...] = jnp.concatenate(mcols, axis=1)


def _expand(qp, reft, maskq, woffp, boffp, wattnp, battnp, selh, base_row):
    grid = QP // QB
    wspec = pl.BlockSpec((QB, IDX_PER_Q), lambda i: (i, 0))
    wshape = jax.ShapeDtypeStruct((QP, IDX_PER_Q), jnp.float32)
    return pl.pallas_call(
        _expand_body,
        grid=(grid,),
        in_specs=[
            pl.BlockSpec((QB, EMBED), lambda i: (i, 0)),
            pl.BlockSpec((QB, 768), lambda i: (i, 0)),
            pl.BlockSpec((QB, 24), lambda i: (i, 0)),
            pl.BlockSpec((EMBED, 512), lambda i: (0, 0)),
            pl.BlockSpec((1, 512), lambda i: (0, 0)),
            pl.BlockSpec((EMBED, EMBED), lambda i: (0, 0)),
            pl.BlockSpec((1, EMBED), lambda i: (0, 0)),
            pl.BlockSpec((HEADS, EMBED), lambda i: (0, 0)),
            pl.BlockSpec((1, 64), lambda i: (0, 0)),
        ],
        out_specs=[
            pl.BlockSpec((QB, IDX_PER_Q), lambda i: (i, 0)),
            wspec, wspec, wspec, wspec,
            pl.BlockSpec((QB, CAMS), lambda i: (i, 0)),
        ],
        out_shape=[
            jax.ShapeDtypeStruct((QP, IDX_PER_Q), jnp.int32),
            wshape, wshape, wshape, wshape,
            jax.ShapeDtypeStruct((QP, CAMS), jnp.float32),
        ],
    )(qp, reft, maskq, woffp, boffp, wattnp, battnp, selh, base_row)


def _sc_gather(v_t, idx_q, w0q, w1q, w2q, w3q):
    mesh = plsc.VectorSubcoreMesh(core_axis_name="c", subcore_axis_name="s")
    BIG = 128
    NBIG = IDX_PER_Q // BIG
    DEPTH = 6
    WPAD = IDX_PER_Q + 16

    @functools.partial(
        pl.kernel,
        mesh=mesh,
        out_type=jax.ShapeDtypeStruct((QP, EMBED), jnp.float32),
        scratch_types=(
            [pltpu.VMEM((IDX_PER_Q,), jnp.int32)] * 2
            + [pltpu.VMEM((WPAD,), jnp.float32)] * 8
            + [pltpu.VMEM((BIG, 128), jnp.float32)] * DEPTH
            + [pltpu.VMEM((EMBED,), jnp.float32)]
            + [pltpu.SemaphoreType.DMA] * (DEPTH + 2)
        ),
    )
    def body(v_hbm, idx_hbm, w0_hbm, w1_hbm, w2_hbm, w3_hbm, out_hbm,
             idx_v0, idx_v1, wa0, wa1, wa2, wa3, wb0, wb1, wb2, wb3,
             b0, b1, b2, b3, b4, b5, out_v, g0, g1, g2, g3, g4, g5,
             si0, si1):
        wid = lax.axis_index("s") * 2 + lax.axis_index("c")
        q0 = wid * QPW
        bufs = (b0, b1, b2, b3, b4, b5)
        gsems = (g0, g1, g2, g3, g4, g5)
        whbms = (w0_hbm, w1_hbm, w2_hbm, w3_hbm)
        isets = ((idx_v0, (wa0, wa1, wa2, wa3), si0),
                 (idx_v1, (wb0, wb1, wb2, wb3), si1))
        zero = jnp.zeros((16,), jnp.float32)

        def process_big(wvs, bc, buf):
            for g in range(BIG // 64):
                goff = (bc * 2 + g) * 64
                roff = g * 64

                def head_body(hh, carry):
                    woff = goff + hh * 8
                    wv = [wvs[t][pl.ds(woff, 16)] for t in range(4)]
                    a = [zero] * 8
                    for k in range(8):
                        r = roff + hh * 8 + k
                        for t in range(4):
                            w = wv[t][k]
                            a[2 * t] = a[2 * t] \
                                + w * buf[r, pl.ds(32 * t, 16)]
                            a[2 * t + 1] = a[2 * t + 1] \
                                + w * buf[r, pl.ds(32 * t + 16, 16)]
                    acc0 = a[0] + a[2] + a[4] + a[6]
                    acc1 = a[1] + a[3] + a[5] + a[7]
                    s0 = hh * HD
                    out_v[pl.ds(s0, 16)] = out_v[pl.ds(s0, 16)] + acc0
                    out_v[pl.ds(s0 + 16, 16)] = \
                        out_v[pl.ds(s0 + 16, 16)] + acc1
                    return carry

                lax.fori_loop(0, HEADS, head_body, 0)

        def fetch(gq, iset):
            idx_v, wvs, sem = iset
            pltpu.async_copy(idx_hbm.at[gq], idx_v, sem)
            for t in range(4):
                pltpu.async_copy(whbms[t].at[gq],
                                 wvs[t].at[pl.ds(0, IDX_PER_Q)], sem)

        def drain(iset):
            idx_v, wvs, sem = iset
            pltpu.make_async_copy(idx_hbm.at[0], idx_v, sem).wait()
            for t in range(4):
                pltpu.make_async_copy(
                    whbms[t].at[0], wvs[t].at[pl.ds(0, IDX_PER_Q)],
                    sem).wait()

        def do_q(gq, iset):
            idx_v, wvs, _ = iset
            for j in range(EMBED // 16):
                out_v[pl.ds(j * 16, 16)] = zero
            handles = []
            for bc in range(NBIG):
                handles.append(pltpu.async_copy(
                    v_hbm.at[idx_v.at[pl.ds(bc * BIG, BIG)]],
                    bufs[bc % DEPTH], gsems[bc % DEPTH]))
                if bc >= DEPTH - 1:
                    handles[bc - DEPTH + 1].wait()
                    process_big(wvs, bc - DEPTH + 1,
                                bufs[(bc - DEPTH + 1) % DEPTH])
            for bc in range(NBIG - DEPTH + 1, NBIG):
                handles[bc].wait()
                process_big(wvs, bc, bufs[bc % DEPTH])
            pltpu.sync_copy(out_v, out_hbm.at[gq])

        fetch(q0, isets[0])

        def per_pair(i, carry):
            qa = q0 + 2 * i
            drain(isets[0])
            fetch(qa + 1, isets[1])
            do_q(qa, isets[0])
            drain(isets[1])
            fetch(jnp.minimum(qa + 2, QP - 1), isets[0])
            do_q(qa + 1, isets[1])
            return carry

        lax.fori_loop(0, QPW // 2, per_pair, 0)
        drain(isets[0])

    return body(v_t, idx_q, w0q, w1q, w2q, w3q)


def _outproj_body(s_ref, m_ref, q_ref, w_ref, b_ref, o_ref):
    cnt = jnp.maximum(jnp.sum(m_ref[...], axis=1, keepdims=True), 1.0)
    s = s_ref[...] / cnt
    o_ref[...] = jnp.dot(s, w_ref[...],
                         preferred_element_type=jnp.float32) \
        + b_ref[...] + q_ref[...]


def _outproj(slots, mq, qpad, w, b):
    blk = 512
    grid = QP // blk
    return pl.pallas_call(
        _outproj_body,
        grid=(grid,),
        in_specs=[
            pl.BlockSpec((blk, EMBED), lambda i: (i, 0)),
            pl.BlockSpec((blk, CAMS), lambda i: (i, 0)),
            pl.BlockSpec((blk, EMBED), lambda i: (i, 0)),
            pl.BlockSpec((EMBED, EMBED), lambda i: (0, 0)),
            pl.BlockSpec((1, EMBED), lambda i: (0, 0)),
        ],
        out_specs=pl.BlockSpec((blk, EMBED), lambda i: (i, 0)),
        out_shape=jax.ShapeDtypeStruct((QP, EMBED), jnp.float32),
    )(slots, mq, qpad, w, b)


def kernel(query, key, value, reference_points_cam, bev_mask, spatial_shapes,
           level_start_index, W_off, b_off, W_attn, b_attn, W_val, b_val,
           W_out, b_out):
    q2 = query[0]

    vproj = _vproj(value, W_val, b_val.reshape(1, EMBED))
    vp2 = jnp.pad(vproj, ((0, 0), (PPAD, PPAD), (0, 0)))
    v4 = _pack(vp2).reshape(CAMS * HEADS * PLANE, 4 * HD)

    pad = QP - NQ
    qp = jnp.pad(q2, ((0, pad), (0, 0)))
    rpc = reference_points_cam[:, 0]
    reft = jnp.pad(
        jnp.tile(rpc.transpose(1, 3, 0, 2), (1, 1, 1, 16)).reshape(NQ, 768),
        ((0, pad), (0, 0)))
    maskq = jnp.pad(
        bev_mask[:, 0].astype(jnp.float32).transpose(1, 0, 2).reshape(NQ, 24),
        ((0, pad), (0, 0)))
    woffp = W_off.reshape(EMBED, HEADS, LEVELS, POINTS, 2) \
        .transpose(0, 4, 2, 1, 3).reshape(EMBED, 512)
    boffp = b_off.reshape(HEADS, LEVELS, POINTS, 2) \
        .transpose(3, 1, 0, 2).reshape(1, 512)
    wattnp = W_attn.reshape(EMBED, HEADS, LEVELS, POINTS) \
        .transpose(0, 2, 1, 3).reshape(EMBED, EMBED)
    battnp = b_attn.reshape(HEADS, LEVELS, POINTS) \
        .transpose(1, 0, 2).reshape(1, EMBED)
    cols = jnp.arange(EMBED, dtype=jnp.int32)
    selh = ((cols[None, :] // POINTS) % HEADS
            == jnp.arange(HEADS, dtype=jnp.int32)[:, None]) \
        .astype(jnp.float32)
    base_row = (jnp.repeat(jnp.arange(HEADS, dtype=jnp.int32), POINTS)
                * PLANE).reshape(1, 64)

    idx_q, w0, w1, w2, w3, m_q = _expand(
        qp, reft, maskq, woffp, boffp, wattnp, battnp, selh, base_row)

    slots = _sc_gather(v4, idx_q, w0, w1, w2, w3)

    out = _outproj(slots, m_q, qp, W_out, b_out.reshape(1, EMBED))
    return out[:NQ].reshape(1, NQ, EMBED)
